# baseline (speedup 1.0000x reference)
import math

import jax
import jax.numpy as jnp
from jax import lax
from jax.experimental import pallas as pl
from jax.experimental.pallas import tpu as pltpu
from jax.experimental.pallas import tpu_sc as plsc

VOCAB = 1000000
D_MODEL = 64
DPAD = 128
SCALE = math.sqrt(D_MODEL)

NC = 2
NS = 16
NW = NC * NS

BATCH = 4096
SEQ = 200
TOKENS = BATCH * SEQ
IW = TOKENS // NW
C = 128
NSTEP = IW // C

PREP_BLK = 4096


def _prep_body(t_ref, w_ref):
    w_ref[:, :D_MODEL] = t_ref[...].T * SCALE


def _lookup_body(table_hbm, idx_hbm, out_hbm,
                 idx_v, gb0, gb1, gb2, ob0, ob1, g0, g1, g2, s0, s1):
    gbuf = (gb0, gb1, gb2)
    obuf = (ob0, ob1)
    gsem = (g0, g1, g2)
    ssem = (s0, s1)

    wid = lax.axis_index("s") * NC + lax.axis_index("c")
    base = wid * (IW // 2)

    pltpu.sync_copy(idx_hbm.at[wid], idx_v)

    def issue_gather(k, b):
        pltpu.async_copy(table_hbm.at[idx_v.at[pl.ds(k * C, C)]],
                         gbuf[b], gsem[b])

    def wait_gather(k, b):
        pltpu.make_async_copy(table_hbm.at[idx_v.at[pl.ds(k * C, C)]],
                              gbuf[b], gsem[b]).wait()

    def issue_store(k, t):
        pltpu.async_copy(obuf[t],
                         out_hbm.at[pl.ds(base + k * (C // 2), C // 2)],
                         ssem[t])

    def wait_store(k, t):
        pltpu.make_async_copy(obuf[t],
                              out_hbm.at[pl.ds(base + k * (C // 2), C // 2)],
                              ssem[t]).wait()

    def compact(b, t):
        src = gbuf[b]
        dst = obuf[t]

        @plsc.parallel_loop(0, C // 2, step=1, unroll=2)
        def _s(p):
            for half in range(2):
                for q in range(D_MODEL // 16):
                    v = src[2 * p + half, pl.ds(16 * q, 16)]
                    dst[p, pl.ds(64 * half + 16 * q, 16)] = v * SCALE

    def body(k, skip_wait_store=False, skip_issue_gather=False):
        b = k % 3
        t = k % 2
        wait_gather(k, b)
        if not skip_wait_store:
            wait_store(k - 2, t)
        compact(b, t)
        issue_store(k, t)
        if not skip_issue_gather:
            issue_gather(k + 2, (k + 2) % 3)

    issue_gather(0, 0)
    issue_gather(1, 1)
    body(0, skip_wait_store=True)
    body(1, skip_wait_store=True)

    @pl.loop(2, NSTEP - 6, step=6)
    def steady(kk):
        for u in range(6):
            k = kk + u
            b = (2 + u) % 3
            t = (2 + u) % 2
            wait_gather(k, b)
            wait_store(k - 2, t)
            compact(b, t)
            issue_store(k, t)
            issue_gather(k + 2, (2 + u + 2) % 3)

    for k in range(NSTEP - 6, NSTEP):
        body(k, skip_issue_gather=(k + 2 >= NSTEP))

    wait_store(NSTEP - 2, (NSTEP - 2) % 2)
    wait_store(NSTEP - 1, (NSTEP - 1) % 2)


@jax.jit
def _emb(x, table):
    mesh = plsc.VectorSubcoreMesh(core_axis_name="c", subcore_axis_name="s")

    wtab = jnp.concatenate(
        [table, jnp.zeros((VOCAB, DPAD - D_MODEL), jnp.float32)], axis=1)
    idx = x.reshape(NW, IW)

    out = pl.kernel(
        _lookup_body,
        out_type=jax.ShapeDtypeStruct((TOKENS // 2, DPAD), jnp.float32),
        mesh=mesh,
        scratch_types=(
            [pltpu.VMEM((IW,), jnp.int32)]
            + [pltpu.VMEM((C, DPAD), jnp.float32)] * 3
            + [pltpu.VMEM((C // 2, DPAD), jnp.float32)] * 2
            + [pltpu.SemaphoreType.DMA] * 5
        ),
        compiler_params=pltpu.CompilerParams(use_tc_tiling_on_sc=True),
    )(wtab, idx)

    return out.reshape(BATCH, SEQ, D_MODEL)


def kernel(x, table):
    return _emb(x, table)


# speedup vs baseline: 1.2475x; 1.2475x over previous
import math

import jax
import jax.numpy as jnp
from jax import lax
from jax.experimental import pallas as pl
from jax.experimental.pallas import tpu as pltpu
from jax.experimental.pallas import tpu_sc as plsc

VOCAB = 1000000
D_MODEL = 64
DPAD = 128
SCALE = math.sqrt(D_MODEL)

NC = 2
NS = 16
NW = NC * NS

BATCH = 4096
SEQ = 200
TOKENS = BATCH * SEQ
IW = TOKENS // NW
C = 128
NSTEP = IW // C


def _lookup_body(table_hbm, idx_hbm, out_hbm,
                 idx_v, gb0, gb1, gb2, ob0, ob1, g0, g1, g2, s0, s1):
    gbuf = (gb0, gb1, gb2)
    obuf = (ob0, ob1)
    gsem = (g0, g1, g2)
    ssem = (s0, s1)

    wid = lax.axis_index("s") * NC + lax.axis_index("c")
    base = wid * IW

    pltpu.sync_copy(idx_hbm.at[wid], idx_v)

    def issue_gather(k, b):
        pltpu.async_copy(table_hbm.at[idx_v.at[pl.ds(k * C, C)]],
                         gbuf[b], gsem[b])

    def wait_gather(k, b):
        pltpu.make_async_copy(table_hbm.at[idx_v.at[pl.ds(k * C, C)]],
                              gbuf[b], gsem[b]).wait()

    def issue_store(k, t):
        pltpu.async_copy(obuf[t], out_hbm.at[pl.ds(base + k * C, C)],
                         ssem[t])

    def wait_store(k, t):
        pltpu.make_async_copy(obuf[t], out_hbm.at[pl.ds(base + k * C, C)],
                              ssem[t]).wait()

    def scale(b, t):
        src = gbuf[b]
        dst = obuf[t]

        @plsc.parallel_loop(0, C, step=1, unroll=2)
        def _s(r):
            for q in range(D_MODEL // 16):
                v = src[r, pl.ds(16 * q, 16)]
                dst[r, pl.ds(16 * q, 16)] = v * SCALE

    def body(k, skip_wait_store=False, skip_issue_gather=False):
        b = k % 3
        t = k % 2
        wait_gather(k, b)
        if not skip_wait_store:
            wait_store(k - 2, t)
        scale(b, t)
        issue_store(k, t)
        if not skip_issue_gather:
            issue_gather(k + 2, (k + 2) % 3)

    issue_gather(0, 0)
    issue_gather(1, 1)
    body(0, skip_wait_store=True)
    body(1, skip_wait_store=True)

    @pl.loop(2, NSTEP - 6, step=6)
    def steady(kk):
        for u in range(6):
            k = kk + u
            b = (2 + u) % 3
            t = (2 + u) % 2
            wait_gather(k, b)
            wait_store(k - 2, t)
            scale(b, t)
            issue_store(k, t)
            issue_gather(k + 2, (2 + u + 2) % 3)

    for k in range(NSTEP - 6, NSTEP):
        body(k, skip_issue_gather=(k + 2 >= NSTEP))

    wait_store(NSTEP - 2, (NSTEP - 2) % 2)
    wait_store(NSTEP - 1, (NSTEP - 1) % 2)


@jax.jit
def _emb(x, table):
    mesh = plsc.VectorSubcoreMesh(core_axis_name="c", subcore_axis_name="s")

    wtab = jnp.pad(table, ((0, 0), (0, DPAD - D_MODEL)))
    idx = x.reshape(NW, IW)

    out = pl.kernel(
        _lookup_body,
        out_type=jax.ShapeDtypeStruct((TOKENS, D_MODEL), jnp.float32),
        mesh=mesh,
        scratch_types=(
            [pltpu.VMEM((IW,), jnp.int32)]
            + [pltpu.VMEM((C, DPAD), jnp.float32)] * 3
            + [pltpu.VMEM((C, D_MODEL), jnp.float32)] * 2
            + [pltpu.SemaphoreType.DMA] * 5
        ),
    )(wtab, idx)

    return out.reshape(BATCH, SEQ, D_MODEL)


def kernel(x, table):
    return _emb(x, table)
